# Initial kernel scaffold; baseline (speedup 1.0000x reference)
import functools

import jax
import jax.numpy as jnp
from jax import lax
from jax.experimental import pallas as pl
from jax.experimental.pallas import tpu as pltpu
from jax.experimental.pallas import tpu_sc as plsc

_TABLE = 100000
_D = 1024
_PRIMES = (17, 31, 53, 79)
_NC = 2
_NS = 16
_NW = _NC * _NS
_CHUNK = 16


def _sc_gather_sum(ids, table):
    n = ids.shape[0]
    n_per_w = n // _NW
    n_chunks = n_per_w // _CHUNK
    mesh = plsc.VectorSubcoreMesh(
        core_axis_name="c", subcore_axis_name="s",
        num_cores=_NC, num_subcores=_NS,
    )

    @functools.partial(
        pl.kernel,
        out_type=jax.ShapeDtypeStruct((n, _D), jnp.float32),
        mesh=mesh,
        scratch_types=[
            pltpu.VMEM((n_per_w,), jnp.int32),
            pltpu.VMEM((4 * _CHUNK,), jnp.int32),
            pltpu.VMEM((3 * _CHUNK,), jnp.int32),
            pltpu.VMEM((4 * _CHUNK, _D), jnp.float32),
            pltpu.VMEM((_CHUNK, _D), jnp.float32),
            pltpu.SemaphoreType.DMA,
        ],
    )
    def kern(ids_hbm, table_hbm, out_hbm, ids_v, idx_v, dst_v, rows_v, acc_v, sem):
        wid = lax.axis_index("s") * _NC + lax.axis_index("c")
        base = wid * n_per_w
        pltpu.sync_copy(ids_hbm.at[pl.ds(base, n_per_w)], ids_v)
        lane = lax.iota(jnp.int32, _CHUNK)
        for h in range(3):
            dst_v[pl.ds(h * _CHUNK, _CHUNK)] = lane

        def body(ci, carry):
            ids16 = ids_v[pl.ds(ci * _CHUNK, _CHUNK)]
            for h in range(4):
                idx_v[pl.ds(h * _CHUNK, _CHUNK)] = (ids16 * _PRIMES[h]) % _TABLE
            pltpu.async_copy(table_hbm.at[idx_v], rows_v, sem).wait()
            pltpu.sync_copy(rows_v.at[pl.ds(0, _CHUNK)], acc_v)
            pltpu.sync_copy(rows_v.at[pl.ds(_CHUNK, 3 * _CHUNK)],
                            acc_v.at[dst_v], add=True)
            pltpu.sync_copy(acc_v, out_hbm.at[pl.ds(base + ci * _CHUNK, _CHUNK)])
            return carry

        lax.fori_loop(0, n_chunks, body, 0)

    return kern(ids, table)


def _tc_dense(hidden, msum, wh, wm, w2, b1, b2, wmerge, bmerge):
    n = hidden.shape[0]
    tb = 512
    grid = (n // tb,)

    def body(hid_ref, sum_ref, wh_ref, wm_ref, w2_ref, b1_ref, b2_ref,
             wmg_ref, bm_ref, out_ref):
        hid = hid_ref[...]
        sm = sum_ref[...]
        pre = jnp.dot(hid.astype(jnp.bfloat16), wh_ref[...],
                      preferred_element_type=jnp.float32)
        pre += jnp.dot(sm.astype(jnp.bfloat16), wm_ref[...],
                       preferred_element_type=jnp.float32)
        pre += b1_ref[...]
        h = jnp.maximum(pre, 0.0)
        g = jnp.sum(h * w2_ref[...], axis=1, keepdims=True) + b2_ref[...]
        g = jax.nn.sigmoid(g)
        gated = (sm * g).astype(jnp.bfloat16)
        out = jnp.dot(gated, wmg_ref[...], preferred_element_type=jnp.float32)
        out_ref[...] = hid + out + bm_ref[...]

    return pl.pallas_call(
        body,
        grid=grid,
        in_specs=[
            pl.BlockSpec((tb, _D), lambda i: (i, 0)),
            pl.BlockSpec((tb, _D), lambda i: (i, 0)),
            pl.BlockSpec((_D, _D), lambda i: (0, 0)),
            pl.BlockSpec((_D, _D), lambda i: (0, 0)),
            pl.BlockSpec((1, _D), lambda i: (0, 0)),
            pl.BlockSpec((1, _D), lambda i: (0, 0)),
            pl.BlockSpec((1, 1), lambda i: (0, 0)),
            pl.BlockSpec((_D, _D), lambda i: (0, 0)),
            pl.BlockSpec((1, _D), lambda i: (0, 0)),
        ],
        out_specs=pl.BlockSpec((tb, _D), lambda i: (i, 0)),
        out_shape=jax.ShapeDtypeStruct((n, _D), jnp.float32),
    )(hidden, msum, wh, wm, w2, b1, b2, wmerge, bmerge)


def kernel(hidden_states, input_ids, memory_table, gate_w1, gate_b1,
           gate_w2, gate_b2, merge_w, merge_b):
    b, s, d = hidden_states.shape
    n = b * s
    ids = input_ids.reshape(n)
    msum = _sc_gather_sum(ids, memory_table)

    wh = jnp.transpose(gate_w1[:, :d]).astype(jnp.bfloat16)
    wm = (0.25 * jnp.transpose(gate_w1[:, d:])).astype(jnp.bfloat16)
    wmerge = (0.25 * jnp.transpose(merge_w)).astype(jnp.bfloat16)

    out = _tc_dense(
        hidden_states.reshape(n, d), msum, wh, wm,
        gate_w2, gate_b1.reshape(1, d), gate_b2.reshape(1, 1),
        wmerge, merge_b.reshape(1, d),
    )
    return out.reshape(b, s, d)


# speedup vs baseline: 2.0798x; 2.0798x over previous
import functools

import jax
import jax.numpy as jnp
from jax import lax
from jax.experimental import pallas as pl
from jax.experimental.pallas import tpu as pltpu
from jax.experimental.pallas import tpu_sc as plsc

_TABLE = 100000
_D = 1024
_PRIMES = (17, 31, 53, 79)
_NC = 2
_NS = 16
_NW = _NC * _NS
_CHUNK = 16


def _sc_gather_sum(ids, table):
    n = ids.shape[0]
    n_per_w = n // _NW
    n_chunks = n_per_w // _CHUNK
    mesh = plsc.VectorSubcoreMesh(
        core_axis_name="c", subcore_axis_name="s",
        num_cores=_NC, num_subcores=_NS,
    )

    @functools.partial(
        pl.kernel,
        out_type=jax.ShapeDtypeStruct((n, _D), jnp.float32),
        mesh=mesh,
        scratch_types=[
            pltpu.VMEM((n_per_w,), jnp.int32),
            pltpu.VMEM((4 * _CHUNK,), jnp.int32),
            pltpu.VMEM((4 * _CHUNK, _D), jnp.float32),
            pltpu.VMEM((_CHUNK, _D), jnp.float32),
            pltpu.SemaphoreType.DMA,
        ],
    )
    def kern(ids_hbm, table_hbm, out_hbm, ids_v, idx_v, rows_v, acc_v, sem):
        wid = lax.axis_index("s") * _NC + lax.axis_index("c")
        base = wid * n_per_w
        pltpu.sync_copy(ids_hbm.at[pl.ds(base, n_per_w)], ids_v)

        def body(ci, carry):
            ids16 = ids_v[pl.ds(ci * _CHUNK, _CHUNK)]
            for h in range(4):
                idx_v[pl.ds(h * _CHUNK, _CHUNK)] = (ids16 * _PRIMES[h]) % _TABLE
            pltpu.async_copy(table_hbm.at[idx_v], rows_v, sem).wait()

            def reduce_tok(t, c):
                for j in range(_D // 16):
                    s = pl.ds(j * 16, 16)
                    acc_v[t, s] = ((rows_v[t, s] + rows_v[_CHUNK + t, s])
                                   + (rows_v[2 * _CHUNK + t, s]
                                      + rows_v[3 * _CHUNK + t, s]))
                return c

            lax.fori_loop(0, _CHUNK, reduce_tok, 0)
            pltpu.sync_copy(acc_v, out_hbm.at[pl.ds(base + ci * _CHUNK, _CHUNK)])
            return carry

        lax.fori_loop(0, n_chunks, body, 0)

    return kern(ids, table)


def _tc_dense(hidden, msum, wh, wm, w2, b1, b2, wmerge, bmerge):
    n = hidden.shape[0]
    tb = 512
    grid = (n // tb,)

    def body(hid_ref, sum_ref, wh_ref, wm_ref, w2_ref, b1_ref, b2_ref,
             wmg_ref, bm_ref, out_ref):
        hid = hid_ref[...]
        sm = sum_ref[...]
        pre = jnp.dot(hid.astype(jnp.bfloat16), wh_ref[...],
                      preferred_element_type=jnp.float32)
        pre += jnp.dot(sm.astype(jnp.bfloat16), wm_ref[...],
                       preferred_element_type=jnp.float32)
        pre += b1_ref[...]
        h = jnp.maximum(pre, 0.0)
        g = jnp.sum(h * w2_ref[...], axis=1, keepdims=True) + b2_ref[...]
        g = jax.nn.sigmoid(g)
        gated = (sm * g).astype(jnp.bfloat16)
        out = jnp.dot(gated, wmg_ref[...], preferred_element_type=jnp.float32)
        out_ref[...] = hid + out + bm_ref[...]

    return pl.pallas_call(
        body,
        grid=grid,
        in_specs=[
            pl.BlockSpec((tb, _D), lambda i: (i, 0)),
            pl.BlockSpec((tb, _D), lambda i: (i, 0)),
            pl.BlockSpec((_D, _D), lambda i: (0, 0)),
            pl.BlockSpec((_D, _D), lambda i: (0, 0)),
            pl.BlockSpec((1, _D), lambda i: (0, 0)),
            pl.BlockSpec((1, _D), lambda i: (0, 0)),
            pl.BlockSpec((1, 1), lambda i: (0, 0)),
            pl.BlockSpec((_D, _D), lambda i: (0, 0)),
            pl.BlockSpec((1, _D), lambda i: (0, 0)),
        ],
        out_specs=pl.BlockSpec((tb, _D), lambda i: (i, 0)),
        out_shape=jax.ShapeDtypeStruct((n, _D), jnp.float32),
    )(hidden, msum, wh, wm, w2, b1, b2, wmerge, bmerge)


def kernel(hidden_states, input_ids, memory_table, gate_w1, gate_b1,
           gate_w2, gate_b2, merge_w, merge_b):
    b, s, d = hidden_states.shape
    n = b * s
    ids = input_ids.reshape(n)
    msum = _sc_gather_sum(ids, memory_table)

    wh = jnp.transpose(gate_w1[:, :d]).astype(jnp.bfloat16)
    wm = (0.25 * jnp.transpose(gate_w1[:, d:])).astype(jnp.bfloat16)
    wmerge = (0.25 * jnp.transpose(merge_w)).astype(jnp.bfloat16)

    out = _tc_dense(
        hidden_states.reshape(n, d), msum, wh, wm,
        gate_w2, gate_b1.reshape(1, d), gate_b2.reshape(1, 1),
        wmerge, merge_b.reshape(1, d),
    )
    return out.reshape(b, s, d)
